# baseline (speedup 1.0000x reference)
import functools

import jax
import jax.numpy as jnp
from jax import lax
from jax.experimental import pallas as pl
from jax.experimental.pallas import tpu as pltpu
from jax.experimental.pallas import tpu_sc as plsc

N = 10000
E = 320000
NFEAT = 128
NHID = 128
NCLASS = 64

NC = 2
NS = 16
NW = NC * NS
EPW = E // NW
K = 128
EPW_P = 10240
C = EPW_P // K
N_PAD = 10240
ROWS_PER_TILE = N_PAD // NS
ZR = 64


def _make_spmm(F):
    mesh = plsc.VectorSubcoreMesh(core_axis_name="c", subcore_axis_name="s")

    @functools.partial(
        pl.kernel,
        mesh=mesh,
        out_type=jax.ShapeDtypeStruct((NC, N_PAD, F), jnp.float32),
        compiler_params=pltpu.CompilerParams(use_tc_tiling_on_sc=False),
        scratch_types=[
            pltpu.VMEM_SHARED((N_PAD, F), jnp.float32),
            pltpu.VMEM((4, 3, K), jnp.int32),
            pltpu.VMEM((2, K, F), jnp.float32),
            pltpu.VMEM((ZR, F), jnp.float32),
            pltpu.SemaphoreType.DMA((2,)),
            pltpu.SemaphoreType.DMA((2,)),
            pltpu.SemaphoreType.DMA((2,)),
            pltpu.SemaphoreType.DMA,
        ],
    )
    def spmm(support_hbm, comb_hbm, out_hbm,
             acc, ebuf, rows_v, zeros_v, gsem, ssem, esem, zsem):
        cid = lax.axis_index("c")
        sid = lax.axis_index("s")
        wid = sid * NC + cid

        for r in range(ZR):
            for j in range(F // 16):
                zeros_v[r, pl.ds(j * 16, 16)] = jnp.zeros((16,), jnp.float32)
        base = sid * ROWS_PER_TILE
        NZ = ROWS_PER_TILE // ZR

        def zissue(i, _):
            pltpu.async_copy(zeros_v, acc.at[pl.ds(base + i * ZR, ZR)], zsem)
            return 0
        lax.fori_loop(0, NZ, zissue, 0)

        def zdrain(i, _):
            pltpu.make_async_copy(zeros_v, acc.at[pl.ds(base, ZR)], zsem).wait()
            return 0
        lax.fori_loop(0, NZ, zdrain, 0)
        plsc.subcore_barrier()

        pltpu.sync_copy(comb_hbm.at[wid, 0], ebuf.at[0])
        pltpu.sync_copy(comb_hbm.at[wid, 1], ebuf.at[1])
        pltpu.async_copy(support_hbm.at[ebuf.at[0, 0]], rows_v.at[0],
                         gsem.at[0])
        pltpu.async_copy(comb_hbm.at[wid, 2], ebuf.at[2], esem.at[0])
        pltpu.async_copy(comb_hbm.at[wid, 3], ebuf.at[3], esem.at[1])

        def chunk(c, _):
            p = c % 2
            i4 = c % 4
            pn = (c + 1) % 2
            i4n = (c + 1) % 4

            @pl.when(c + 1 < C)
            def _():
                @pl.when(c + 1 >= 2)
                def _():
                    pltpu.make_async_copy(
                        comb_hbm.at[wid, c + 1], ebuf.at[i4n],
                        esem.at[pn]).wait()
                pltpu.async_copy(support_hbm.at[ebuf.at[i4n, 0]],
                                 rows_v.at[pn], gsem.at[pn])

            @pl.when((c >= 2) & (c + 2 < C))
            def _():
                pltpu.async_copy(comb_hbm.at[wid, c + 2], ebuf.at[(c + 2) % 4],
                                 esem.at[c % 2])

            pltpu.make_async_copy(support_hbm.at[ebuf.at[i4, 0]],
                                  rows_v.at[p], gsem.at[p]).wait()

            def scale(kk, _):
                w16 = lax.bitcast_convert_type(
                    ebuf[i4, 2, pl.ds(kk * 16, 16)], jnp.float32)
                for i in range(16):
                    w = jnp.full((16,), w16[i], jnp.float32)
                    k = kk * 16 + i
                    for j in range(F // 16):
                        sl = pl.ds(j * 16, 16)
                        rows_v[p, k, sl] = rows_v[p, k, sl] * w
                return 0
            lax.fori_loop(0, K // 16, scale, 0)

            pltpu.async_copy(rows_v.at[p], acc.at[ebuf.at[i4, 1]],
                             ssem.at[p], add=True)
            pltpu.make_async_copy(rows_v.at[p], acc.at[ebuf.at[i4, 1]],
                                  ssem.at[p]).wait()
            return 0
        lax.fori_loop(0, C, chunk, 0)

        plsc.subcore_barrier()
        pltpu.sync_copy(acc.at[pl.ds(base, ROWS_PER_TILE)],
                        out_hbm.at[cid, pl.ds(base, ROWS_PER_TILE)])

    return spmm


_spmm_hid = _make_spmm(NHID)
_spmm_cls = _make_spmm(NCLASS)


_BM = 1000


def _mm1_body(x_ref, w_ref, o_ref):
    o_ref[...] = jnp.dot(x_ref[...], w_ref[...],
                         preferred_element_type=jnp.float32)


def _mm1(x, W1):
    return pl.pallas_call(
        _mm1_body,
        grid=(N // _BM,),
        in_specs=[
            pl.BlockSpec((_BM, NFEAT), lambda i: (i, 0)),
            pl.BlockSpec((NFEAT, NHID), lambda i: (0, 0)),
        ],
        out_specs=pl.BlockSpec((_BM, NHID), lambda i: (i, 0)),
        out_shape=jax.ShapeDtypeStruct((N, NHID), jnp.float32),
    )(x, W1)


def _mid_body(p_ref, b_ref, w_ref, o_ref):
    h = jax.nn.relu(p_ref[0] + p_ref[1] + b_ref[...])
    o_ref[...] = jnp.dot(h, w_ref[...], preferred_element_type=jnp.float32)


def _mid(partials, b1, W2):
    return pl.pallas_call(
        _mid_body,
        grid=(N // _BM,),
        in_specs=[
            pl.BlockSpec((NC, _BM, NHID), lambda i: (0, i, 0)),
            pl.BlockSpec((1, NHID), lambda i: (0, 0)),
            pl.BlockSpec((NHID, NCLASS), lambda i: (0, 0)),
        ],
        out_specs=pl.BlockSpec((_BM, NCLASS), lambda i: (i, 0)),
        out_shape=jax.ShapeDtypeStruct((N, NCLASS), jnp.float32),
    )(partials, b1.reshape(1, NHID), W2)


def _final_body(p_ref, b_ref, o_ref):
    o = p_ref[0] + p_ref[1] + b_ref[...]
    m = jnp.max(o, axis=1, keepdims=True)
    e = jnp.exp(o - m)
    s = jnp.sum(e, axis=1, keepdims=True)
    o_ref[...] = o - m - jnp.log(s)


def _final(partials, b2):
    return pl.pallas_call(
        _final_body,
        grid=(N // _BM,),
        in_specs=[
            pl.BlockSpec((NC, _BM, NCLASS), lambda i: (0, i, 0)),
            pl.BlockSpec((1, NCLASS), lambda i: (0, 0)),
        ],
        out_specs=pl.BlockSpec((_BM, NCLASS), lambda i: (i, 0)),
        out_shape=jax.ShapeDtypeStruct((N, NCLASS), jnp.float32),
    )(partials, b2.reshape(1, NCLASS))


def kernel(x, edge_index, edge_weight, W1, b1, W2, b2):
    pad = EPW_P - EPW

    def prep(a):
        return jnp.pad(a.reshape(NW, EPW), ((0, 0), (0, pad))).reshape(NW, C, K)
    comb = jnp.stack(
        [prep(edge_index[0]), prep(edge_index[1]),
         prep(lax.bitcast_convert_type(edge_weight, jnp.int32))], axis=2)

    support1 = _mm1(x, W1)
    p1 = _spmm_hid(support1, comb)
    support2 = _mid(p1, b1, W2)
    p2 = _spmm_cls(support2, comb)
    return _final(p2, b2)


# speedup vs baseline: 1.0724x; 1.0724x over previous
import functools

import jax
import jax.numpy as jnp
from jax import lax
from jax.experimental import pallas as pl
from jax.experimental.pallas import tpu as pltpu
from jax.experimental.pallas import tpu_sc as plsc

N = 10000
E = 320000
NFEAT = 128
NHID = 128
NCLASS = 64

NC = 2
NS = 16
NW = NC * NS
EPW = E // NW
K = 128
EPW_P = 10240
C = EPW_P // K
N_PAD = 10240
ROWS_PER_TILE = N_PAD // NS
ZR = 64


def _make_spmm(F):
    mesh = plsc.VectorSubcoreMesh(core_axis_name="c", subcore_axis_name="s")

    @functools.partial(
        pl.kernel,
        mesh=mesh,
        out_type=jax.ShapeDtypeStruct((NC, N_PAD, F), jnp.float32),
        compiler_params=pltpu.CompilerParams(use_tc_tiling_on_sc=False),
        scratch_types=[
            pltpu.VMEM_SHARED((N_PAD, F), jnp.float32),
            pltpu.VMEM((4, 3, K), jnp.int32),
            pltpu.VMEM((2, K, F), jnp.float32),
            pltpu.VMEM((ZR, F), jnp.float32),
            pltpu.SemaphoreType.DMA((2,)),
            pltpu.SemaphoreType.DMA((2,)),
            pltpu.SemaphoreType.DMA((2,)),
            pltpu.SemaphoreType.DMA,
        ],
    )
    def spmm(support_hbm, comb_hbm, out_hbm,
             acc, ebuf, rows_v, zeros_v, gsem, ssem, esem, zsem):
        cid = lax.axis_index("c")
        sid = lax.axis_index("s")
        wid = sid * NC + cid

        for r in range(ZR):
            for j in range(F // 16):
                zeros_v[r, pl.ds(j * 16, 16)] = jnp.zeros((16,), jnp.float32)
        base = sid * ROWS_PER_TILE
        NZ = ROWS_PER_TILE // ZR

        def zissue(i, _):
            pltpu.async_copy(zeros_v, acc.at[pl.ds(base + i * ZR, ZR)], zsem)
            return 0
        lax.fori_loop(0, NZ, zissue, 0)

        def zdrain(i, _):
            pltpu.make_async_copy(zeros_v, acc.at[pl.ds(base, ZR)], zsem).wait()
            return 0
        lax.fori_loop(0, NZ, zdrain, 0)
        plsc.subcore_barrier()

        def chunk(c, _):
            pltpu.sync_copy(comb_hbm.at[wid, c], ebuf.at[0])
            pltpu.sync_copy(support_hbm.at[ebuf.at[0, 0]], rows_v.at[0])

            def scale(kk, _):
                w16 = lax.bitcast_convert_type(
                    ebuf[0, 2, pl.ds(kk * 16, 16)], jnp.float32)
                for i in range(16):
                    w = jnp.full((16,), w16[i], jnp.float32)
                    k = kk * 16 + i
                    for j in range(F // 16):
                        sl = pl.ds(j * 16, 16)
                        rows_v[0, k, sl] = rows_v[0, k, sl] * w
                return 0
            lax.fori_loop(0, K // 16, scale, 0)

            pltpu.sync_copy(rows_v.at[0], acc.at[ebuf.at[0, 1]], add=True)
            return 0
        lax.fori_loop(0, C, chunk, 0)

        plsc.subcore_barrier()
        pltpu.sync_copy(acc.at[pl.ds(base, ROWS_PER_TILE)],
                        out_hbm.at[cid, pl.ds(base, ROWS_PER_TILE)])

    return spmm


_spmm_hid = _make_spmm(NHID)
_spmm_cls = _make_spmm(NCLASS)


_BM = 1000


def _mm1_body(x_ref, w_ref, o_ref):
    o_ref[...] = jnp.dot(x_ref[...], w_ref[...],
                         preferred_element_type=jnp.float32)


def _mm1(x, W1):
    return pl.pallas_call(
        _mm1_body,
        grid=(N // _BM,),
        in_specs=[
            pl.BlockSpec((_BM, NFEAT), lambda i: (i, 0)),
            pl.BlockSpec((NFEAT, NHID), lambda i: (0, 0)),
        ],
        out_specs=pl.BlockSpec((_BM, NHID), lambda i: (i, 0)),
        out_shape=jax.ShapeDtypeStruct((N, NHID), jnp.float32),
    )(x, W1)


def _mid_body(p_ref, b_ref, w_ref, o_ref):
    h = jax.nn.relu(p_ref[0] + p_ref[1] + b_ref[...])
    o_ref[...] = jnp.dot(h, w_ref[...], preferred_element_type=jnp.float32)


def _mid(partials, b1, W2):
    return pl.pallas_call(
        _mid_body,
        grid=(N // _BM,),
        in_specs=[
            pl.BlockSpec((NC, _BM, NHID), lambda i: (0, i, 0)),
            pl.BlockSpec((1, NHID), lambda i: (0, 0)),
            pl.BlockSpec((NHID, NCLASS), lambda i: (0, 0)),
        ],
        out_specs=pl.BlockSpec((_BM, NCLASS), lambda i: (i, 0)),
        out_shape=jax.ShapeDtypeStruct((N, NCLASS), jnp.float32),
    )(partials, b1.reshape(1, NHID), W2)


def _final_body(p_ref, b_ref, o_ref):
    o = p_ref[0] + p_ref[1] + b_ref[...]
    m = jnp.max(o, axis=1, keepdims=True)
    e = jnp.exp(o - m)
    s = jnp.sum(e, axis=1, keepdims=True)
    o_ref[...] = o - m - jnp.log(s)


def _final(partials, b2):
    return pl.pallas_call(
        _final_body,
        grid=(N // _BM,),
        in_specs=[
            pl.BlockSpec((NC, _BM, NCLASS), lambda i: (0, i, 0)),
            pl.BlockSpec((1, NCLASS), lambda i: (0, 0)),
        ],
        out_specs=pl.BlockSpec((_BM, NCLASS), lambda i: (i, 0)),
        out_shape=jax.ShapeDtypeStruct((N, NCLASS), jnp.float32),
    )(partials, b2.reshape(1, NCLASS))


def kernel(x, edge_index, edge_weight, W1, b1, W2, b2):
    pad = EPW_P - EPW

    def prep(a):
        return jnp.pad(a.reshape(NW, EPW), ((0, 0), (0, pad))).reshape(NW, C, K)
    comb = jnp.stack(
        [prep(edge_index[0]), prep(edge_index[1]),
         prep(lax.bitcast_convert_type(edge_weight, jnp.int32))], axis=2)

    support1 = _mm1(x, W1)
    p1 = _spmm_hid(support1, comb)
    support2 = _mid(p1, b1, W2)
    p2 = _spmm_cls(support2, comb)
    return _final(p2, b2)


# speedup vs baseline: 1.6136x; 1.5047x over previous
import functools

import jax
import jax.numpy as jnp
from jax import lax
from jax.experimental import pallas as pl
from jax.experimental.pallas import tpu as pltpu
from jax.experimental.pallas import tpu_sc as plsc

N = 10000
E = 320000
NFEAT = 128
NHID = 128
NCLASS = 64

NC = 2
NS = 16
NW = NC * NS
EPW = E // NW
K = 80
C = EPW // K
N_PAD = 10240
ROWS_PER_TILE = N_PAD // NS
ZR = 8
SHIFT = 14


def _make_spmm(F):
    mesh = plsc.VectorSubcoreMesh(core_axis_name="c", subcore_axis_name="s")

    @functools.partial(
        pl.kernel,
        mesh=mesh,
        out_type=jax.ShapeDtypeStruct((NC, N_PAD, F), jnp.float32),
        compiler_params=pltpu.CompilerParams(use_tc_tiling_on_sc=False),
        scratch_types=[
            pltpu.VMEM_SHARED((N_PAD, F), jnp.float32),
            pltpu.VMEM((C, K), jnp.int32),
            pltpu.VMEM((EPW,), jnp.float32),
            pltpu.VMEM((2, K), jnp.int32),
            pltpu.VMEM((2, K), jnp.int32),
            pltpu.VMEM((2, K, F), jnp.float32),
            pltpu.VMEM((ZR, F), jnp.float32),
            pltpu.SemaphoreType.DMA((2,)),
            pltpu.SemaphoreType.DMA,
        ],
    )
    def spmm(support_hbm, pk_hbm, w_hbm, out_hbm,
             acc, pk_v, w_v, sbuf, dbuf, rows_v, zeros_v, gsem, zsem):
        cid = lax.axis_index("c")
        sid = lax.axis_index("s")
        wid = sid * NC + cid

        for r in range(ZR):
            for j in range(F // 16):
                zeros_v[r, pl.ds(j * 16, 16)] = jnp.zeros((16,), jnp.float32)
        base = sid * ROWS_PER_TILE
        NZ = ROWS_PER_TILE // ZR

        def zissue(i, _):
            pltpu.async_copy(zeros_v, acc.at[pl.ds(base + i * ZR, ZR)], zsem)
            return 0
        lax.fori_loop(0, NZ, zissue, 0)

        def zdrain(i, _):
            pltpu.make_async_copy(zeros_v, acc.at[pl.ds(base, ZR)], zsem).wait()
            return 0
        lax.fori_loop(0, NZ, zdrain, 0)
        plsc.subcore_barrier()

        pltpu.sync_copy(pk_hbm.at[wid], pk_v)
        pltpu.sync_copy(w_hbm.at[pl.ds(wid * EPW, EPW)], w_v)

        def unpack(c, b):
            for kk in range(K // 16):
                sl = pl.ds(kk * 16, 16)
                v = pk_v[c, sl]
                sbuf[b, sl] = jnp.bitwise_and(v, (1 << SHIFT) - 1)
                dbuf[b, sl] = lax.shift_right_logical(v, SHIFT)

        unpack(0, 0)
        pltpu.async_copy(support_hbm.at[sbuf.at[0]], rows_v.at[0], gsem.at[0])

        def chunk(c, _):
            p = c % 2
            pn = (c + 1) % 2

            @pl.when(c + 1 < C)
            def _():
                unpack(c + 1, pn)
                pltpu.async_copy(support_hbm.at[sbuf.at[pn]], rows_v.at[pn],
                                 gsem.at[pn])

            pltpu.make_async_copy(support_hbm.at[sbuf.at[p]], rows_v.at[p],
                                  gsem.at[p]).wait()

            def scale(kk, _):
                w16 = w_v[pl.ds(c * K + kk * 16, 16)]
                for i in range(16):
                    w = jnp.full((16,), w16[i], jnp.float32)
                    k = kk * 16 + i
                    for j in range(F // 16):
                        sl = pl.ds(j * 16, 16)
                        rows_v[p, k, sl] = rows_v[p, k, sl] * w
                return 0
            lax.fori_loop(0, K // 16, scale, 0)

            pltpu.sync_copy(rows_v.at[p], acc.at[dbuf.at[p]], add=True)
            return 0
        lax.fori_loop(0, C, chunk, 0)

        plsc.subcore_barrier()
        pltpu.sync_copy(acc.at[pl.ds(base, ROWS_PER_TILE)],
                        out_hbm.at[cid, pl.ds(base, ROWS_PER_TILE)])

    return spmm


_spmm_hid = _make_spmm(NHID)
_spmm_cls = _make_spmm(NCLASS)


_BM = 1000


def _mm1_body(x_ref, w_ref, o_ref):
    o_ref[...] = jnp.dot(x_ref[...], w_ref[...],
                         preferred_element_type=jnp.float32)


def _mm1(x, W1):
    return pl.pallas_call(
        _mm1_body,
        grid=(N // _BM,),
        in_specs=[
            pl.BlockSpec((_BM, NFEAT), lambda i: (i, 0)),
            pl.BlockSpec((NFEAT, NHID), lambda i: (0, 0)),
        ],
        out_specs=pl.BlockSpec((_BM, NHID), lambda i: (i, 0)),
        out_shape=jax.ShapeDtypeStruct((N, NHID), jnp.float32),
    )(x, W1)


def _mid_body(p_ref, b_ref, w_ref, o_ref):
    h = jax.nn.relu(p_ref[0] + p_ref[1] + b_ref[...])
    o_ref[...] = jnp.dot(h, w_ref[...], preferred_element_type=jnp.float32)


def _mid(partials, b1, W2):
    return pl.pallas_call(
        _mid_body,
        grid=(N // _BM,),
        in_specs=[
            pl.BlockSpec((NC, _BM, NHID), lambda i: (0, i, 0)),
            pl.BlockSpec((1, NHID), lambda i: (0, 0)),
            pl.BlockSpec((NHID, NCLASS), lambda i: (0, 0)),
        ],
        out_specs=pl.BlockSpec((_BM, NCLASS), lambda i: (i, 0)),
        out_shape=jax.ShapeDtypeStruct((N, NCLASS), jnp.float32),
    )(partials, b1.reshape(1, NHID), W2)


def _final_body(p_ref, b_ref, o_ref):
    o = p_ref[0] + p_ref[1] + b_ref[...]
    m = jnp.max(o, axis=1, keepdims=True)
    e = jnp.exp(o - m)
    s = jnp.sum(e, axis=1, keepdims=True)
    o_ref[...] = o - m - jnp.log(s)


def _final(partials, b2):
    return pl.pallas_call(
        _final_body,
        grid=(N // _BM,),
        in_specs=[
            pl.BlockSpec((NC, _BM, NCLASS), lambda i: (0, i, 0)),
            pl.BlockSpec((1, NCLASS), lambda i: (0, 0)),
        ],
        out_specs=pl.BlockSpec((_BM, NCLASS), lambda i: (i, 0)),
        out_shape=jax.ShapeDtypeStruct((N, NCLASS), jnp.float32),
    )(partials, b2.reshape(1, NCLASS))


def kernel(x, edge_index, edge_weight, W1, b1, W2, b2):
    packed = (edge_index[0] + (edge_index[1] << SHIFT)).reshape(NW, C, K)

    support1 = _mm1(x, W1)
    p1 = _spmm_hid(support1, packed, edge_weight)
    support2 = _mid(p1, b1, W2)
    p2 = _spmm_cls(support2, packed, edge_weight)
    return _final(p2, b2)


# speedup vs baseline: 1.6152x; 1.0009x over previous
import functools

import jax
import jax.numpy as jnp
from jax import lax
from jax.experimental import pallas as pl
from jax.experimental.pallas import tpu as pltpu
from jax.experimental.pallas import tpu_sc as plsc

N = 10000
E = 320000
NFEAT = 128
NHID = 128
NCLASS = 64

NC = 2
NS = 16
NW = NC * NS
EPW = E // NW
K = 80
C = EPW // K
N_PAD = 10240
ROWS_PER_TILE = N_PAD // NS
ZR = 8
SHIFT = 14


def _make_spmm(F):
    mesh = plsc.VectorSubcoreMesh(core_axis_name="c", subcore_axis_name="s")

    @functools.partial(
        pl.kernel,
        mesh=mesh,
        out_type=jax.ShapeDtypeStruct((NC, N_PAD, F), jnp.float32),
        compiler_params=pltpu.CompilerParams(use_tc_tiling_on_sc=False),
        scratch_types=[
            pltpu.VMEM_SHARED((N_PAD, F), jnp.float32),
            pltpu.VMEM((C, K), jnp.int32),
            pltpu.VMEM((EPW,), jnp.float32),
            pltpu.VMEM((2, K), jnp.int32),
            pltpu.VMEM((2, K), jnp.int32),
            pltpu.VMEM((2, K, F), jnp.float32),
            pltpu.VMEM((ZR, F), jnp.float32),
            pltpu.SemaphoreType.DMA((2,)),
            pltpu.SemaphoreType.DMA((2,)),
            pltpu.SemaphoreType.DMA,
        ],
    )
    def spmm(support_hbm, pk_hbm, w_hbm, out_hbm,
             acc, pk_v, w_v, sbuf, dbuf, rows_v, zeros_v, gsem, ssem, zsem):
        cid = lax.axis_index("c")
        sid = lax.axis_index("s")
        wid = sid * NC + cid

        for r in range(ZR):
            for j in range(F // 16):
                zeros_v[r, pl.ds(j * 16, 16)] = jnp.zeros((16,), jnp.float32)
        base = sid * ROWS_PER_TILE
        NZ = ROWS_PER_TILE // ZR

        def zissue(i, _):
            pltpu.async_copy(zeros_v, acc.at[pl.ds(base + i * ZR, ZR)], zsem)
            return 0
        lax.fori_loop(0, NZ, zissue, 0)

        def zdrain(i, _):
            pltpu.make_async_copy(zeros_v, acc.at[pl.ds(base, ZR)], zsem).wait()
            return 0
        lax.fori_loop(0, NZ, zdrain, 0)
        plsc.subcore_barrier()

        pltpu.sync_copy(pk_hbm.at[wid], pk_v)
        pltpu.sync_copy(w_hbm.at[pl.ds(wid * EPW, EPW)], w_v)

        def unpack(c, b):
            for kk in range(K // 16):
                sl = pl.ds(kk * 16, 16)
                v = pk_v[c, sl]
                sbuf[b, sl] = jnp.bitwise_and(v, (1 << SHIFT) - 1)
                dbuf[b, sl] = lax.shift_right_logical(v, SHIFT)

        unpack(0, 0)
        pltpu.async_copy(support_hbm.at[sbuf.at[0]], rows_v.at[0], gsem.at[0])

        def chunk(c, _):
            p = c % 2
            pn = (c + 1) % 2

            @pl.when((c >= 1) & (c + 1 < C))
            def _():
                pltpu.make_async_copy(rows_v.at[pn], acc.at[dbuf.at[pn]],
                                      ssem.at[pn]).wait()

            @pl.when(c + 1 < C)
            def _():
                unpack(c + 1, pn)
                pltpu.async_copy(support_hbm.at[sbuf.at[pn]], rows_v.at[pn],
                                 gsem.at[pn])

            pltpu.make_async_copy(support_hbm.at[sbuf.at[p]], rows_v.at[p],
                                  gsem.at[p]).wait()

            def scale(kk, _):
                w16 = w_v[pl.ds(c * K + kk * 16, 16)]
                for i in range(16):
                    w = jnp.full((16,), w16[i], jnp.float32)
                    k = kk * 16 + i
                    for j in range(F // 16):
                        sl = pl.ds(j * 16, 16)
                        rows_v[p, k, sl] = rows_v[p, k, sl] * w
                return 0
            lax.fori_loop(0, K // 16, scale, 0)

            pltpu.async_copy(rows_v.at[p], acc.at[dbuf.at[p]], ssem.at[p],
                             add=True)
            return 0
        lax.fori_loop(0, C, chunk, 0)

        for cc in (C - 2, C - 1):
            pltpu.make_async_copy(rows_v.at[cc % 2], acc.at[dbuf.at[cc % 2]],
                                  ssem.at[cc % 2]).wait()

        plsc.subcore_barrier()
        pltpu.sync_copy(acc.at[pl.ds(base, ROWS_PER_TILE)],
                        out_hbm.at[cid, pl.ds(base, ROWS_PER_TILE)])

    return spmm


_spmm_hid = _make_spmm(NHID)
_spmm_cls = _make_spmm(NCLASS)


_BM = 1000


def _mm1_body(x_ref, w_ref, o_ref):
    o_ref[...] = jnp.dot(x_ref[...], w_ref[...],
                         preferred_element_type=jnp.float32)


def _mm1(x, W1):
    return pl.pallas_call(
        _mm1_body,
        grid=(N // _BM,),
        in_specs=[
            pl.BlockSpec((_BM, NFEAT), lambda i: (i, 0)),
            pl.BlockSpec((NFEAT, NHID), lambda i: (0, 0)),
        ],
        out_specs=pl.BlockSpec((_BM, NHID), lambda i: (i, 0)),
        out_shape=jax.ShapeDtypeStruct((N, NHID), jnp.float32),
    )(x, W1)


def _mid_body(p_ref, b_ref, w_ref, o_ref):
    h = jax.nn.relu(p_ref[0] + p_ref[1] + b_ref[...])
    o_ref[...] = jnp.dot(h, w_ref[...], preferred_element_type=jnp.float32)


def _mid(partials, b1, W2):
    return pl.pallas_call(
        _mid_body,
        grid=(N // _BM,),
        in_specs=[
            pl.BlockSpec((NC, _BM, NHID), lambda i: (0, i, 0)),
            pl.BlockSpec((1, NHID), lambda i: (0, 0)),
            pl.BlockSpec((NHID, NCLASS), lambda i: (0, 0)),
        ],
        out_specs=pl.BlockSpec((_BM, NCLASS), lambda i: (i, 0)),
        out_shape=jax.ShapeDtypeStruct((N, NCLASS), jnp.float32),
    )(partials, b1.reshape(1, NHID), W2)


def _final_body(p_ref, b_ref, o_ref):
    o = p_ref[0] + p_ref[1] + b_ref[...]
    m = jnp.max(o, axis=1, keepdims=True)
    e = jnp.exp(o - m)
    s = jnp.sum(e, axis=1, keepdims=True)
    o_ref[...] = o - m - jnp.log(s)


def _final(partials, b2):
    return pl.pallas_call(
        _final_body,
        grid=(N // _BM,),
        in_specs=[
            pl.BlockSpec((NC, _BM, NCLASS), lambda i: (0, i, 0)),
            pl.BlockSpec((1, NCLASS), lambda i: (0, 0)),
        ],
        out_specs=pl.BlockSpec((_BM, NCLASS), lambda i: (i, 0)),
        out_shape=jax.ShapeDtypeStruct((N, NCLASS), jnp.float32),
    )(partials, b2.reshape(1, NCLASS))


def kernel(x, edge_index, edge_weight, W1, b1, W2, b2):
    packed = (edge_index[0] + (edge_index[1] << SHIFT)).reshape(NW, C, K)

    support1 = _mm1(x, W1)
    p1 = _spmm_hid(support1, packed, edge_weight)
    support2 = _mid(p1, b1, W2)
    p2 = _spmm_cls(support2, packed, edge_weight)
    return _final(p2, b2)
